# Initial kernel scaffold; baseline (speedup 1.0000x reference)
import functools

import jax
import jax.numpy as jnp
from jax import lax
from jax.experimental import pallas as pl
from jax.experimental.pallas import tpu as pltpu
from jax.experimental.pallas import tpu_sc as plsc

BATCH = 16384
BAG = 32
E = 32

NC = 2
NS = 16
L = 16
NW = NC * NS

ROWS_PER_W = BATCH // NW
IDX_PER_W = ROWS_PER_W * BAG
G = 128
NG = IDX_PER_W // G
BAGS_PER_G = G // BAG
NBUF = 4


def _bag_body(fi_hbm, tab_hbm, out_hbm, idx_v, rows_v, out_v, sem):
  wid = lax.axis_index("s") * NC + lax.axis_index("c")
  pltpu.sync_copy(fi_hbm.at[wid], idx_v)

  for b in range(NBUF):
    pltpu.async_copy(tab_hbm.at[idx_v.at[b]], rows_v.at[b], sem)

  def outer(i, carry):
    g0 = i * NBUF
    for b in range(NBUF):
      g = g0 + b
      pltpu.make_async_copy(tab_hbm.at[idx_v.at[0]], rows_v.at[b], sem).wait()
      for bag in range(BAGS_PER_G):
        for h in range(2):
          acc = rows_v[b, bag * BAG, pl.ds(h * L, L)]
          for r in range(1, BAG):
            acc = acc + rows_v[b, bag * BAG + r, pl.ds(h * L, L)]
          out_v[g * BAGS_PER_G + bag, pl.ds(h * L, L)] = acc

      @pl.when(g + NBUF < NG)
      def _():
        pltpu.async_copy(tab_hbm.at[idx_v.at[g + NBUF]], rows_v.at[b], sem)

    return carry

  lax.fori_loop(0, NG // NBUF, outer, 0, unroll=False)
  pltpu.sync_copy(out_v, out_hbm.at[pl.ds(wid * ROWS_PER_W, ROWS_PER_W)])


@jax.jit
def _embedding_bag(fi3, emb_table):
  mesh = plsc.VectorSubcoreMesh(
      core_axis_name="c", subcore_axis_name="s", num_cores=NC, num_subcores=NS
  )
  return pl.kernel(
      _bag_body,
      out_type=jax.ShapeDtypeStruct((BATCH, E), jnp.float32),
      mesh=mesh,
      scratch_types=[
          pltpu.VMEM((NG, G), jnp.int32),
          pltpu.VMEM((NBUF, G, E), jnp.float32),
          pltpu.VMEM((ROWS_PER_W, E), jnp.float32),
          pltpu.SemaphoreType.DMA,
      ],
  )(fi3, emb_table)


def _mlp_body(x_ref, w1_ref, b1_ref, w2_ref, b2_ref, w3_ref, b3_ref, o_ref):
  x = x_ref[...]
  h = jnp.maximum(
      jnp.dot(x, w1_ref[...], preferred_element_type=jnp.float32) + b1_ref[...],
      0.0,
  )
  h = jnp.maximum(
      jnp.dot(h, w2_ref[...], preferred_element_type=jnp.float32) + b2_ref[...],
      0.0,
  )
  o_ref[...] = jnp.sum(h * w3_ref[...], axis=1, keepdims=True) + b3_ref[...]


@functools.partial(jax.jit, static_argnames=("tb",))
def _mlp(x, W1, b1, W2, b2, w3_row, b3, tb=2048):
  grid = BATCH // tb
  full = lambda s: pl.BlockSpec(s, lambda i: (0, 0))
  return pl.pallas_call(
      _mlp_body,
      grid=(grid,),
      in_specs=[
          pl.BlockSpec((tb, E), lambda i: (i, 0)),
          full(W1.shape),
          full(b1.shape),
          full(W2.shape),
          full(b2.shape),
          full(w3_row.shape),
          full(b3.shape),
      ],
      out_specs=pl.BlockSpec((tb, 1), lambda i: (i, 0)),
      out_shape=jax.ShapeDtypeStruct((BATCH, 1), jnp.float32),
  )(x, W1, b1, W2, b2, w3_row, b3)


def kernel(feature_indices, emb_table, W1, b1, W2, b2, W3, b3):
  fi3 = feature_indices.reshape(NW, NG, G)
  bags = _embedding_bag(fi3, emb_table)
  return _mlp(
      bags,
      W1,
      b1.reshape(1, -1),
      W2,
      b2.reshape(1, -1),
      W3.reshape(1, -1),
      b3.reshape(1, 1),
  )


# speedup vs baseline: 4.1682x; 4.1682x over previous
import functools

import jax
import jax.numpy as jnp
from jax import lax
from jax.experimental import pallas as pl
from jax.experimental.pallas import tpu as pltpu
from jax.experimental.pallas import tpu_sc as plsc

BATCH = 16384
BAG = 32
E = 32

NC = 2
NS = 16
L = 16
NW = NC * NS

ROWS_PER_W = BATCH // NW
IDX_PER_W = ROWS_PER_W * BAG
G = 128
NG = IDX_PER_W // G
BAGS_PER_G = G // BAG
NBUF = 4


def _bag_body(fi_hbm, tab_hbm, out_hbm, idx_v, rows_v, out_v, sem):
  wid = lax.axis_index("s") * NC + lax.axis_index("c")
  pltpu.sync_copy(fi_hbm.at[wid], idx_v)

  def remap(j, carry):
    for k in range(G // L):
      v = idx_v[j, pl.ds(k * L, L)]
      idx_v[j, pl.ds(k * L, L)] = (
          (v & (-RB)) + ((v & 2047) << 2) + ((v >> 11) & 3)
      )
    return carry

  lax.fori_loop(0, NG, remap, 0, unroll=False)

  for b in range(NBUF):
    pltpu.async_copy(tab_hbm.at[idx_v.at[b]], rows_v.at[b], sem)

  def outer(i, carry):
    g0 = i * NBUF
    for b in range(NBUF):
      g = g0 + b
      pltpu.make_async_copy(tab_hbm.at[idx_v.at[0]], rows_v.at[b], sem).wait()
      for bag in range(BAGS_PER_G):
        for h in range(2):
          vals = [
              rows_v[b, bag * BAG + r, pl.ds(h * L, L)]
              + rows_v[b, bag * BAG + r + 1, pl.ds(h * L, L)]
              for r in range(0, BAG, 2)
          ]
          while len(vals) > 1:
            vals = [vals[j] + vals[j + 1] for j in range(0, len(vals), 2)]
          out_v[g * BAGS_PER_G + bag, pl.ds(h * L, L)] = vals[0]

      @pl.when(g + NBUF < NG)
      def _():
        pltpu.async_copy(tab_hbm.at[idx_v.at[g + NBUF]], rows_v.at[b], sem)

    return carry

  lax.fori_loop(0, NG // NBUF, outer, 0, unroll=False)
  pltpu.sync_copy(out_v, out_hbm.at[pl.ds(wid * ROWS_PER_W, ROWS_PER_W)])


@jax.jit
def _embedding_bag(fi3, emb_table):
  mesh = plsc.VectorSubcoreMesh(
      core_axis_name="c", subcore_axis_name="s", num_cores=NC, num_subcores=NS
  )
  return pl.kernel(
      _bag_body,
      out_type=jax.ShapeDtypeStruct((BATCH, E), jnp.float32),
      mesh=mesh,
      scratch_types=[
          pltpu.VMEM((NG, G), jnp.int32),
          pltpu.VMEM((NBUF, G, E), jnp.float32),
          pltpu.VMEM((ROWS_PER_W, E), jnp.float32),
          pltpu.SemaphoreType.DMA,
      ],
      compiler_params=pltpu.CompilerParams(use_tc_tiling_on_sc=False),
  )(fi3, emb_table)


RB = 8192
NBLK = (1000000 + RB - 1) // RB
PAD_ROWS = NBLK * RB


def _retile_body(x_ref, o_ref):
  q = RB // 4
  xx = jnp.concatenate(
      [x_ref[:, pl.ds(a * q, q)] for a in range(4)], axis=0
  )
  o_ref[...] = xx.T


@jax.jit
def _retile(tabT):
  return pl.pallas_call(
      _retile_body,
      grid=(NBLK,),
      in_specs=[pl.BlockSpec((32, RB), lambda i: (0, i))],
      out_specs=pl.BlockSpec((RB // 4, 128), lambda i: (i, 0)),
      out_shape=jax.ShapeDtypeStruct((PAD_ROWS // 4, 128), jnp.float32),
  )(tabT)


def _mlp_body(x_ref, w1_ref, b1_ref, w2_ref, b2_ref, w3_ref, b3_ref, o_ref):
  x = x_ref[...]
  h = jnp.maximum(
      jnp.dot(x, w1_ref[...], preferred_element_type=jnp.float32) + b1_ref[...],
      0.0,
  )
  h = jnp.maximum(
      jnp.dot(h, w2_ref[...], preferred_element_type=jnp.float32) + b2_ref[...],
      0.0,
  )
  o_ref[...] = jnp.sum(h * w3_ref[...], axis=1, keepdims=True) + b3_ref[...]


@functools.partial(jax.jit, static_argnames=("tb",))
def _mlp(x, W1, b1, W2, b2, w3_row, b3, tb=2048):
  grid = BATCH // tb
  full = lambda s: pl.BlockSpec(s, lambda i: (0, 0))
  return pl.pallas_call(
      _mlp_body,
      grid=(grid,),
      in_specs=[
          pl.BlockSpec((tb, E), lambda i: (i, 0)),
          full(W1.shape),
          full(b1.shape),
          full(W2.shape),
          full(b2.shape),
          full(w3_row.shape),
          full(b3.shape),
      ],
      out_specs=pl.BlockSpec((tb, 1), lambda i: (i, 0)),
      out_shape=jax.ShapeDtypeStruct((BATCH, 1), jnp.float32),
  )(x, W1, b1, W2, b2, w3_row, b3)


def kernel(feature_indices, emb_table, W1, b1, W2, b2, W3, b3):
  fi3 = feature_indices.reshape(NW, NG, G)
  tab2 = _retile(emb_table.T)
  bags = _embedding_bag(fi3, tab2.reshape(PAD_ROWS, E))
  return _mlp(
      bags,
      W1,
      b1.reshape(1, -1),
      W2,
      b2.reshape(1, -1),
      W3.reshape(1, -1),
      b3.reshape(1, 1),
  )


# speedup vs baseline: 6.2256x; 1.4936x over previous
import functools

import jax
import jax.numpy as jnp
from jax import lax
from jax.experimental import pallas as pl
from jax.experimental.pallas import tpu as pltpu
from jax.experimental.pallas import tpu_sc as plsc

BATCH = 16384
BAG = 32
E = 32

NC = 2
NS = 16
L = 16
NW = NC * NS

ROWS_PER_W = BATCH // NW
IDX_PER_W = ROWS_PER_W * BAG
G = 128
NG = IDX_PER_W // G
BAGS_PER_G = G // BAG
NBUF = 4


def _bag_body(fi_hbm, tab_hbm, out_hbm, idx_v, rows_v, out_v, sem):
  wid = lax.axis_index("s") * NC + lax.axis_index("c")
  pltpu.sync_copy(fi_hbm.at[pl.ds(wid * NG, NG)], idx_v)

  def remap(j, carry):
    for k in range(G // L):
      v = idx_v[j, pl.ds(k * L, L)]
      idx_v[j, pl.ds(k * L, L)] = (
          (v & (-RB)) + ((v & 8191) << 2) + ((v >> 13) & 3)
      )
    return carry

  lax.fori_loop(0, NG, remap, 0, unroll=False)

  for b in range(NBUF):
    pltpu.async_copy(tab_hbm.at[idx_v.at[b]], rows_v.at[b], sem)

  def outer(i, carry):
    g0 = i * NBUF
    for b in range(NBUF):
      g = g0 + b
      pltpu.make_async_copy(tab_hbm.at[idx_v.at[0]], rows_v.at[b], sem).wait()
      for bag in range(BAGS_PER_G):
        for h in range(2):
          vals = [
              rows_v[b, bag * BAG + r, pl.ds(h * L, L)]
              + rows_v[b, bag * BAG + r + 1, pl.ds(h * L, L)]
              for r in range(0, BAG, 2)
          ]
          while len(vals) > 1:
            vals = [vals[j] + vals[j + 1] for j in range(0, len(vals), 2)]
          out_v[g, pl.ds(bag * BAG + h * L, L)] = vals[0]

      @pl.when(g + NBUF < NG)
      def _():
        pltpu.async_copy(tab_hbm.at[idx_v.at[g + NBUF]], rows_v.at[b], sem)

    return carry

  lax.fori_loop(0, NG // NBUF, outer, 0, unroll=False)
  pltpu.sync_copy(out_v, out_hbm.at[pl.ds(wid * NG, NG)])


@jax.jit
def _embedding_bag(fi2p, emb_table):
  mesh = plsc.VectorSubcoreMesh(
      core_axis_name="c", subcore_axis_name="s", num_cores=NC, num_subcores=NS
  )
  return pl.kernel(
      _bag_body,
      out_type=jax.ShapeDtypeStruct((BATCH // 4, 4 * E), jnp.float32),
      mesh=mesh,
      scratch_types=[
          pltpu.VMEM((NG, G), jnp.int32),
          pltpu.VMEM((NBUF, G, E), jnp.float32),
          pltpu.VMEM((NG, G), jnp.float32),
          pltpu.SemaphoreType.DMA,
      ],
      compiler_params=pltpu.CompilerParams(use_tc_tiling_on_sc=False),
  )(fi2p, emb_table)


RB = 32768
NBLK = (1000000 + RB - 1) // RB
PAD_ROWS = NBLK * RB


def _retile_body(x_ref, o_ref):
  q = RB // 4
  xx = jnp.concatenate(
      [x_ref[:, pl.ds(a * q, q)] for a in range(4)], axis=0
  )
  o_ref[...] = xx.T


@jax.jit
def _retile(tabT):
  return pl.pallas_call(
      _retile_body,
      grid=(NBLK,),
      in_specs=[pl.BlockSpec((32, RB), lambda i: (0, i))],
      out_specs=pl.BlockSpec((RB // 4, 128), lambda i: (i, 0)),
      out_shape=jax.ShapeDtypeStruct((PAD_ROWS // 4, 128), jnp.float32),
  )(tabT)


def _retile_fi_body(x_ref, o_ref):
  xx = jnp.concatenate(
      [x_ref[:, pl.ds(a * 4096, 4096)] for a in range(4)], axis=0
  )
  o_ref[...] = xx.T


@jax.jit
def _retile_fi(fiT):
  return pl.pallas_call(
      _retile_fi_body,
      grid=(1,),
      in_specs=[pl.BlockSpec((32, BATCH), lambda i: (0, 0))],
      out_specs=pl.BlockSpec((BATCH // 4, 128), lambda i: (0, 0)),
      out_shape=jax.ShapeDtypeStruct((BATCH // 4, 128), jnp.int32),
  )(fiT)


def _mlp_body(x_ref, w1_ref, b1_ref, w2_ref, b2_ref, w3_ref, b3_ref, o_ref):
  x = x_ref[...]
  h = jnp.maximum(
      jnp.dot(x, w1_ref[...], preferred_element_type=jnp.float32) + b1_ref[...],
      0.0,
  )
  h = jnp.maximum(
      jnp.dot(h, w2_ref[...], preferred_element_type=jnp.float32) + b2_ref[...],
      0.0,
  )
  o_ref[...] = (
      jnp.dot(h, w3_ref[...], preferred_element_type=jnp.float32) + b3_ref[...]
  )


@functools.partial(jax.jit, static_argnames=("tb",))
def _mlp(x, W1, b1, W2, b2, W3, b3, tb=1024):
  grid = (BATCH // 4) // tb
  eye4 = jnp.eye(4, dtype=jnp.float32)
  w1x = jnp.kron(eye4, W1)
  b1x = jnp.tile(b1, 4).reshape(1, -1)
  w2x = jnp.kron(eye4, W2)
  b2x = jnp.tile(b2, 4).reshape(1, -1)
  w3x = jnp.kron(eye4, W3)
  b3x = jnp.tile(b3, 4).reshape(1, -1)
  full = lambda s: pl.BlockSpec(s, lambda i: (0, 0))
  return pl.pallas_call(
      _mlp_body,
      grid=(grid,),
      in_specs=[
          pl.BlockSpec((tb, 4 * E), lambda i: (i, 0)),
          full(w1x.shape),
          full(b1x.shape),
          full(w2x.shape),
          full(b2x.shape),
          full(w3x.shape),
          full(b3x.shape),
      ],
      out_specs=pl.BlockSpec((tb, 4), lambda i: (i, 0)),
      out_shape=jax.ShapeDtypeStruct((BATCH // 4, 4), jnp.float32),
  )(x, w1x, b1x, w2x, b2x, w3x, b3x)


def kernel(feature_indices, emb_table, W1, b1, W2, b2, W3, b3):
  fi2p = _retile_fi(feature_indices.T)
  tab2 = _retile(emb_table.T)
  bags4 = _embedding_bag(fi2p, tab2.reshape(PAD_ROWS, E))
  return _mlp(bags4, W1, b1, W2, b2, W3, b3).T.reshape(BATCH, 1)


# speedup vs baseline: 6.2989x; 1.0118x over previous
import functools

import jax
import jax.numpy as jnp
from jax import lax
from jax.experimental import pallas as pl
from jax.experimental.pallas import tpu as pltpu
from jax.experimental.pallas import tpu_sc as plsc

BATCH = 16384
BAG = 32
E = 32

NC = 2
NS = 16
L = 16
NW = NC * NS

ROWS_PER_W = BATCH // NW
IDX_PER_W = ROWS_PER_W * BAG
G = 128
NG = IDX_PER_W // G
BAGS_PER_G = G // BAG
NBUF = 4


def _bag_body(fi_hbm, tab_hbm, out_hbm, idx_v, rows_v, out_v, sem):
  wid = lax.axis_index("s") * NC + lax.axis_index("c")
  pltpu.sync_copy(fi_hbm.at[pl.ds(wid * NG, NG)], idx_v)

  def remap(j, carry):
    for k in range(G // L):
      v = idx_v[j, pl.ds(k * L, L)]
      idx_v[j, pl.ds(k * L, L)] = (
          (v & (-RB)) + ((v & 16383) << 2) + ((v >> 14) & 3)
      )
    return carry

  lax.fori_loop(0, NG, remap, 0, unroll=False)

  for b in range(NBUF):
    pltpu.async_copy(tab_hbm.at[idx_v.at[b]], rows_v.at[b], sem)

  def outer(i, carry):
    g0 = i * NBUF
    for b in range(NBUF):
      g = g0 + b
      pltpu.make_async_copy(tab_hbm.at[idx_v.at[0]], rows_v.at[b], sem).wait()
      for bag in range(BAGS_PER_G):
        for h in range(2):
          vals = [
              rows_v[b, bag * BAG + r, pl.ds(h * L, L)]
              + rows_v[b, bag * BAG + r + 1, pl.ds(h * L, L)]
              for r in range(0, BAG, 2)
          ]
          while len(vals) > 1:
            vals = [vals[j] + vals[j + 1] for j in range(0, len(vals), 2)]
          out_v[g, pl.ds(bag * BAG + h * L, L)] = vals[0]

      @pl.when(g + NBUF < NG)
      def _():
        pltpu.async_copy(tab_hbm.at[idx_v.at[g + NBUF]], rows_v.at[b], sem)

    return carry

  lax.fori_loop(0, NG // NBUF, outer, 0, unroll=False)
  pltpu.sync_copy(out_v, out_hbm.at[pl.ds(wid * NG, NG)])


@jax.jit
def _embedding_bag(fi2p, emb_table):
  mesh = plsc.VectorSubcoreMesh(
      core_axis_name="c", subcore_axis_name="s", num_cores=NC, num_subcores=NS
  )
  return pl.kernel(
      _bag_body,
      out_type=jax.ShapeDtypeStruct((BATCH // 4, 4 * E), jnp.float32),
      mesh=mesh,
      scratch_types=[
          pltpu.VMEM((NG, G), jnp.int32),
          pltpu.VMEM((NBUF, G, E), jnp.float32),
          pltpu.VMEM((NG, G), jnp.float32),
          pltpu.SemaphoreType.DMA,
      ],
      compiler_params=pltpu.CompilerParams(use_tc_tiling_on_sc=False),
  )(fi2p, emb_table)


RB = 65536
NBLK = (1000000 + RB - 1) // RB
PAD_ROWS = NBLK * RB


def _retile_body(x_ref, o_ref):
  q = RB // 4
  xx = jnp.concatenate(
      [x_ref[:, pl.ds(a * q, q)] for a in range(4)], axis=0
  )
  o_ref[...] = xx.T


@jax.jit
def _retile(tabT):
  return pl.pallas_call(
      _retile_body,
      grid=(NBLK,),
      in_specs=[pl.BlockSpec((32, RB), lambda i: (0, i))],
      out_specs=pl.BlockSpec((RB // 4, 128), lambda i: (i, 0)),
      out_shape=jax.ShapeDtypeStruct((PAD_ROWS // 4, 128), jnp.float32),
  )(tabT)


def _retile_fi_body(x_ref, o_ref):
  xx = jnp.concatenate(
      [x_ref[:, pl.ds(a * 4096, 4096)] for a in range(4)], axis=0
  )
  o_ref[...] = xx.T


@jax.jit
def _retile_fi(fiT):
  return pl.pallas_call(
      _retile_fi_body,
      grid=(1,),
      in_specs=[pl.BlockSpec((32, BATCH), lambda i: (0, 0))],
      out_specs=pl.BlockSpec((BATCH // 4, 128), lambda i: (0, 0)),
      out_shape=jax.ShapeDtypeStruct((BATCH // 4, 128), jnp.int32),
  )(fiT)


def _mlp_body(x_ref, w1_ref, b1_ref, w2_ref, b2_ref, w3_ref, b3_ref, o_ref):
  x = x_ref[...]
  h = jnp.maximum(
      jnp.dot(x, w1_ref[...], preferred_element_type=jnp.float32) + b1_ref[...],
      0.0,
  )
  h = jnp.maximum(
      jnp.dot(h, w2_ref[...], preferred_element_type=jnp.float32) + b2_ref[...],
      0.0,
  )
  o_ref[...] = (
      jnp.dot(h, w3_ref[...], preferred_element_type=jnp.float32) + b3_ref[...]
  )


@functools.partial(jax.jit, static_argnames=("tb",))
def _mlp(x, W1, b1, W2, b2, W3, b3, tb=1024):
  grid = (BATCH // 4) // tb
  eye4 = jnp.eye(4, dtype=jnp.float32)
  w1x = jnp.kron(eye4, W1)
  b1x = jnp.tile(b1, 4).reshape(1, -1)
  w2x = jnp.kron(eye4, W2)
  b2x = jnp.tile(b2, 4).reshape(1, -1)
  w3x = jnp.kron(eye4, W3)
  b3x = jnp.tile(b3, 4).reshape(1, -1)
  full = lambda s: pl.BlockSpec(s, lambda i: (0, 0))
  return pl.pallas_call(
      _mlp_body,
      grid=(grid,),
      in_specs=[
          pl.BlockSpec((tb, 4 * E), lambda i: (i, 0)),
          full(w1x.shape),
          full(b1x.shape),
          full(w2x.shape),
          full(b2x.shape),
          full(w3x.shape),
          full(b3x.shape),
      ],
      out_specs=pl.BlockSpec((tb, 4), lambda i: (i, 0)),
      out_shape=jax.ShapeDtypeStruct((BATCH // 4, 4), jnp.float32),
  )(x, w1x, b1x, w2x, b2x, w3x, b3x)


def kernel(feature_indices, emb_table, W1, b1, W2, b2, W3, b3):
  fi2p = _retile_fi(feature_indices.T)
  tab2 = _retile(emb_table.T)
  bags4 = _embedding_bag(fi2p, tab2.reshape(PAD_ROWS, E))
  return _mlp(bags4, W1, b1, W2, b2, W3, b3).T.reshape(BATCH, 1)
